# baseline (speedup 1.0000x reference)
import jax
import jax.numpy as jnp
from jax.experimental import pallas as pl
from jax.experimental.pallas import tpu as pltpu


def _csum_kernel(x_ref, o_ref):
    o_ref[0] = jnp.sum(x_ref[0], axis=0)


def kernel(x):
    b, c, h, w = x.shape
    return pl.pallas_call(
        _csum_kernel,
        out_shape=jax.ShapeDtypeStruct((b, h, w), x.dtype),
        grid=(b,),
        in_specs=[pl.BlockSpec((1, c, h, w), lambda i: (i, 0, 0, 0))],
        out_specs=pl.BlockSpec((1, h, w), lambda i: (i, 0, 0)),
        compiler_params=pltpu.CompilerParams(
            dimension_semantics=("parallel",),
            vmem_limit_bytes=64 * 1024 * 1024,
        ),
    )(x)


# speedup vs baseline: 5.8087x; 5.8087x over previous
import jax
import jax.numpy as jnp
from jax.experimental import pallas as pl
from jax.experimental.pallas import tpu as pltpu

_BR = 4096


def _lane_sum_kernel(x_ref, o_ref):
    o_ref[...] = jnp.sum(x_ref[...], axis=-1, keepdims=True)


def kernel(x):
    b, c, h, w = x.shape
    rows = b * h * w
    x2d = jnp.transpose(x, (0, 2, 3, 1)).reshape(rows, c)

    out = pl.pallas_call(
        _lane_sum_kernel,
        out_shape=jax.ShapeDtypeStruct((rows, 1), x.dtype),
        grid=(rows // _BR,),
        in_specs=[pl.BlockSpec((_BR, c), lambda i: (i, 0))],
        out_specs=pl.BlockSpec((_BR, 1), lambda i: (i, 0)),
        compiler_params=pltpu.CompilerParams(
            dimension_semantics=("parallel",),
            vmem_limit_bytes=64 * 1024 * 1024,
        ),
    )(x2d)
    return out.reshape(b, h, w)


# speedup vs baseline: 6.0209x; 1.0365x over previous
import jax
import jax.numpy as jnp
from jax.experimental import pallas as pl
from jax.experimental.pallas import tpu as pltpu

_BR = 8192


def _lane_sum_kernel(x_ref, o_ref):
    o_ref[...] = jnp.sum(x_ref[...], axis=-1, keepdims=True)


def kernel(x):
    b, c, h, w = x.shape
    rows = b * h * w
    x2d = jnp.transpose(x, (0, 2, 3, 1)).reshape(rows, c)

    out = pl.pallas_call(
        _lane_sum_kernel,
        out_shape=jax.ShapeDtypeStruct((rows, 1), x.dtype),
        grid=(rows // _BR,),
        in_specs=[pl.BlockSpec((_BR, c), lambda i: (i, 0))],
        out_specs=pl.BlockSpec((_BR, 1), lambda i: (i, 0)),
        compiler_params=pltpu.CompilerParams(
            dimension_semantics=("parallel",),
            vmem_limit_bytes=64 * 1024 * 1024,
        ),
    )(x2d)
    return out.reshape(b, h, w)
